# Initial kernel scaffold; baseline (speedup 1.0000x reference)
import jax
import jax.numpy as jnp
from jax.experimental import pallas as pl


def kernel(loc_data, conf_data, targets, priors):
    raise NotImplementedError("write your pallas kernel here")


# speedup vs baseline: 31.9929x; 31.9929x over previous
import functools

import jax
import jax.numpy as jnp
from jax.experimental import pallas as pl
from jax.experimental.pallas import tpu as pltpu

THRESHOLD = 0.5
VARIANCES = (0.1, 0.2)
NEGPOS_RATIO = 3.0


def _body(conf_ref, loc_ref, tgt_ref, pri_ref, out_ref,
          lc_ref, npos_ref, acc_ref, *, B, P, C, T):
    b = pl.program_id(0)

    tgt = tgt_ref[0]
    txmin = tgt[:, 0:1]
    tymin = tgt[:, 1:2]
    txmax = tgt[:, 2:3]
    tymax = tgt[:, 3:4]
    tlabel = tgt[:, 4:5]

    pcx = pri_ref[0:1, :]
    pcy = pri_ref[1:2, :]
    pw = pri_ref[2:3, :]
    ph = pri_ref[3:4, :]
    pxmin = pcx - pw / 2.0
    pymin = pcy - ph / 2.0
    pxmax = pcx + pw / 2.0
    pymax = pcy + ph / 2.0

    ix = jnp.clip(jnp.minimum(txmax, pxmax) - jnp.maximum(txmin, pxmin), 0.0, None)
    iy = jnp.clip(jnp.minimum(tymax, pymax) - jnp.maximum(tymin, pymin), 0.0, None)
    inter = ix * iy
    area_t = (txmax - txmin) * (tymax - tymin)
    area_p = (pxmax - pxmin) * (pymax - pymin)
    ov = inter / (area_t + area_p - inter)

    t_iota = jax.lax.broadcasted_iota(jnp.int32, (T, P), 0)
    p_iota = jax.lax.broadcasted_iota(jnp.int32, (T, P), 1)

    btv = jnp.max(ov, axis=0, keepdims=True)
    bti = jnp.min(jnp.where(ov == btv, t_iota, T), axis=0, keepdims=True)
    bpv = jnp.max(ov, axis=1, keepdims=True)
    bpi = jnp.min(jnp.where(ov == bpv, p_iota, P), axis=1, keepdims=True)

    m = bpi == p_iota
    forced = jnp.max(m.astype(jnp.int32), axis=0, keepdims=True) > 0
    ch_t = jnp.max(jnp.where(m, t_iota, -1), axis=0, keepdims=True)
    bti = jnp.where(forced, ch_t, bti)
    btv = jnp.where(forced, 2.0, btv)

    oh = (t_iota == bti).astype(jnp.float32)
    mx0 = jnp.sum(oh * txmin, axis=0, keepdims=True)
    my0 = jnp.sum(oh * tymin, axis=0, keepdims=True)
    mx1 = jnp.sum(oh * txmax, axis=0, keepdims=True)
    my1 = jnp.sum(oh * tymax, axis=0, keepdims=True)
    mlab = jnp.sum(oh * tlabel, axis=0, keepdims=True)

    cls = jnp.where(btv < THRESHOLD, 0.0, mlab)
    cls_i = cls.astype(jnp.int32)
    pos = cls_i > 0

    gx = ((mx0 + mx1) / 2.0 - pcx) / (VARIANCES[0] * pw)
    gy = ((my0 + my1) / 2.0 - pcy) / (VARIANCES[0] * ph)
    gw = jnp.log((mx1 - mx0) / pw + 1e-8) / VARIANCES[1]
    gh = jnp.log((my1 - my0) / ph + 1e-8) / VARIANCES[1]

    loc = loc_ref[0]
    posf = pos.astype(jnp.float32)

    def sl1(d):
        ad = jnp.abs(d)
        return jnp.where(ad < 1.0, 0.5 * ad * ad, ad - 0.5)

    loss_l_b = jnp.sum((sl1(loc[0:1, :] - gx) + sl1(loc[1:2, :] - gy) +
                        sl1(loc[2:3, :] - gw) + sl1(loc[3:4, :] - gh)) * posf)

    conf = conf_ref[0]
    cmax = jnp.max(conf, axis=0, keepdims=True)
    sexp = jnp.sum(jnp.exp(conf - cmax), axis=0, keepdims=True)
    lse = cmax + jnp.log(sexp)
    c_iota = jax.lax.broadcasted_iota(jnp.int32, (C, P), 0)
    gathered = jnp.sum(jnp.where(c_iota == cls_i, conf, 0.0),
                       axis=0, keepdims=True)
    ce = lse - gathered
    loss_c = jnp.where(pos, 0.0, ce)

    npos_b = jnp.sum(posf)
    spce_b = jnp.sum(ce * posf)

    lc_ref[pl.ds(b, 1), :] = loss_c
    npos_ref[pl.ds(b, 1), :] = npos_b.reshape(1, 1)

    @pl.when(b == 0)
    def _init():
        acc_ref[0] = 0.0
        acc_ref[1] = 0.0

    acc_ref[0] += loss_l_b
    acc_ref[1] += spce_b

    @pl.when(b == B - 1)
    def _final():
        lc = lc_ref[...]
        npos = npos_ref[...]
        k = jnp.minimum(NEGPOS_RATIO * npos, float(P - 1))
        k_i = k.astype(jnp.int32)
        bits = jax.lax.bitcast_convert_type(lc, jnp.int32)

        def step(_, carry):
            lo, hi = carry
            mid = lo + (hi - lo) // 2
            cnt = jnp.sum((bits >= mid).astype(jnp.int32), axis=1, keepdims=True)
            ge = cnt >= k_i
            return jnp.where(ge, mid, lo), jnp.where(ge, hi, mid)

        lo0 = jnp.zeros((B, 1), jnp.int32)
        hi0 = jnp.full((B, 1), jnp.int32(0x7F800001))
        lo, _ = jax.lax.fori_loop(0, 31, step, (lo0, hi0))
        tval = jax.lax.bitcast_convert_type(lo, jnp.float32)
        gt = bits > lo
        cnt_gt = jnp.sum(gt.astype(jnp.int32), axis=1, keepdims=True)
        sum_gt = jnp.sum(jnp.where(gt, lc, 0.0), axis=1, keepdims=True)
        S = sum_gt + jnp.where(k_i > cnt_gt,
                               (k_i - cnt_gt).astype(jnp.float32) * tval, 0.0)
        n_total = jnp.sum(npos)
        loss_conf = acc_ref[1] + jnp.sum(S)
        out_ref[:, :] = ((acc_ref[0] + loss_conf) / n_total).reshape(1, 1)


def kernel(loc_data, conf_data, targets, priors):
    B, P, C = conf_data.shape
    T = targets.shape[1]
    conf_r = jnp.transpose(conf_data, (0, 2, 1))
    loc_r = jnp.transpose(loc_data, (0, 2, 1))
    pri_r = priors.T

    out = pl.pallas_call(
        functools.partial(_body, B=B, P=P, C=C, T=T),
        grid=(B,),
        in_specs=[
            pl.BlockSpec((1, C, P), lambda b: (b, 0, 0)),
            pl.BlockSpec((1, 4, P), lambda b: (b, 0, 0)),
            pl.BlockSpec((1, T, 5), lambda b: (b, 0, 0)),
            pl.BlockSpec((4, P), lambda b: (0, 0)),
        ],
        out_specs=pl.BlockSpec((1, 1), lambda b: (0, 0)),
        out_shape=jax.ShapeDtypeStruct((1, 1), jnp.float32),
        scratch_shapes=[
            pltpu.VMEM((B, P), jnp.float32),
            pltpu.VMEM((B, 1), jnp.float32),
            pltpu.SMEM((2,), jnp.float32),
        ],
    )(conf_r, loc_r, targets, pri_r)
    return out[0, 0]
